# baseline (speedup 1.0000x reference)
import functools

import jax
import jax.numpy as jnp
from jax.experimental import pallas as pl
from jax.experimental.pallas import tpu as pltpu
from jax.experimental.pallas import tpu_sc as plsc

_NW = 32
_CHUNK = 200
_H = 4
_BLK = 1600


def _sc_gather(table, idx):
    b = idx.shape[0]
    _, d = table.shape
    b_per_w = b // _NW
    n_chunks = b_per_w // _CHUNK
    n_pairs = n_chunks // 2
    mesh = plsc.VectorSubcoreMesh(core_axis_name="c", subcore_axis_name="s")

    @functools.partial(
        pl.kernel,
        out_type=jax.ShapeDtypeStruct((b, d), table.dtype),
        mesh=mesh,
        scratch_types=[
            pltpu.VMEM((_CHUNK,), jnp.int32),
            pltpu.VMEM((_CHUNK,), jnp.int32),
            pltpu.VMEM((_CHUNK, d), table.dtype),
            pltpu.VMEM((_CHUNK, d), table.dtype),
            pltpu.SemaphoreType.DMA,
            pltpu.SemaphoreType.DMA,
            pltpu.SemaphoreType.DMA,
            pltpu.SemaphoreType.DMA,
        ],
    )
    def gather_kernel(table_hbm, idx_hbm, out_hbm,
                      i0, i1, r0, r1, sg0, sg1, sw0, sw1):
        wid = jax.lax.axis_index("s") * 2 + jax.lax.axis_index("c")
        tile_base = wid * b_per_w

        @pl.loop(0, n_pairs)
        def _(p):
            base0 = tile_base + 2 * p * _CHUNK
            base1 = base0 + _CHUNK

            @pl.when(p > 0)
            def _():
                pltpu.make_async_copy(
                    r0, out_hbm.at[pl.ds(base0 - 2 * _CHUNK, _CHUNK)],
                    sw0).wait()
                pltpu.make_async_copy(
                    r1, out_hbm.at[pl.ds(base1 - 2 * _CHUNK, _CHUNK)],
                    sw1).wait()

            pltpu.sync_copy(idx_hbm.at[pl.ds(base0, _CHUNK)], i0)
            g0 = pltpu.async_copy(table_hbm.at[i0], r0, sg0)
            pltpu.sync_copy(idx_hbm.at[pl.ds(base1, _CHUNK)], i1)
            g1 = pltpu.async_copy(table_hbm.at[i1], r1, sg1)
            g0.wait()
            pltpu.async_copy(r0, out_hbm.at[pl.ds(base0, _CHUNK)], sw0)
            g1.wait()
            pltpu.async_copy(r1, out_hbm.at[pl.ds(base1, _CHUNK)], sw1)

        end0 = tile_base + (n_chunks - 2) * _CHUNK
        pltpu.make_async_copy(
            r0, out_hbm.at[pl.ds(end0, _CHUNK)], sw0).wait()
        pltpu.make_async_copy(
            r1, out_hbm.at[pl.ds(end0 + _CHUNK, _CHUNK)], sw1).wait()

    return gather_kernel(table, idx)


def _band_matrix(d, k):
    i = jnp.arange(d)
    band = (jnp.abs(i[:, None] - i[None, :]) <= (k - 1) // 2)
    return band.astype(jnp.float32) * (1.0 / k)


def _smooth_matrix(d):
    m = jnp.zeros((4 * d, 3 * d), jnp.float32)
    for b, k in enumerate((3, 5, 7)):
        m = m.at[b * d:(b + 1) * d, b * d:(b + 1) * d].set(_band_matrix(d, k))
    return m.astype(jnp.bfloat16).astype(jnp.float32)


def _strip_body(raw_ref, mlo_ref, mhi_ref, out_ref):
    raw = raw_ref[...]
    lo = jax.lax.bitcast_convert_type(raw << 16, jnp.float32)
    hi = jax.lax.bitcast_convert_type(raw & jnp.int32(-65536), jnp.float32)
    out_ref[...] = (
        jnp.dot(lo, mlo_ref[...], preferred_element_type=jnp.float32)
        + jnp.dot(hi, mhi_ref[...], preferred_element_type=jnp.float32))


def _strip_body_alias(raw_ref, mlo_ref, mhi_ref, alias_ref, out_ref):
    del alias_ref
    _strip_body(raw_ref, mlo_ref, mhi_ref, out_ref)


def _smooth_compact(raw_h, mlo, mhi, out_prev, h, b_total):
    bh, dpak = raw_h.shape
    d_in, d_out = mlo.shape
    nb = bh // _BLK
    off = h * nb
    out_spec = pl.BlockSpec((_BLK, d_out), lambda j: (j + off, 0))
    out_shape = jax.ShapeDtypeStruct((b_total, d_out), jnp.float32)
    m_spec = pl.BlockSpec((d_in, d_out), lambda j: (0, 0))
    if out_prev is None:
        return pl.pallas_call(
            _strip_body,
            grid=(nb,),
            in_specs=[pl.BlockSpec((_BLK, dpak), lambda j: (j, 0)),
                      m_spec, m_spec],
            out_specs=out_spec,
            out_shape=out_shape,
        )(raw_h, mlo, mhi)
    return pl.pallas_call(
        _strip_body_alias,
        grid=(nb,),
        in_specs=[pl.BlockSpec((_BLK, dpak), lambda j: (j, 0)),
                  m_spec, m_spec,
                  pl.BlockSpec(memory_space=pltpu.MemorySpace.HBM)],
        out_specs=out_spec,
        out_shape=out_shape,
        input_output_aliases={3: 0},
    )(raw_h, mlo, mhi, out_prev)


def kernel(x, W3, W5, W7):
    bsz, seq = x.shape
    v, d = W3.shape
    b_total = bsz * seq
    b_h = b_total // _H

    tall16 = jnp.concatenate(
        [W3, W5, W7, jnp.zeros((v, d), W3.dtype)], axis=1
    ).astype(jnp.bfloat16)
    tall = jax.lax.bitcast_convert_type(
        tall16.reshape(v, 2 * d, 2), jnp.int32)
    idx = x.reshape(-1).astype(jnp.int32)
    m = _smooth_matrix(d)
    mlo, mhi = m[0::2], m[1::2]

    out = None
    for h in range(_H):
        raw_h = _sc_gather(tall, idx[h * b_h:(h + 1) * b_h])
        out = _smooth_compact(raw_h, mlo, mhi, out, h, b_total)
    return out.reshape(bsz, seq, 3 * d)


# speedup vs baseline: 1.5689x; 1.5689x over previous
import functools

import jax
import jax.numpy as jnp
from jax.experimental import pallas as pl
from jax.experimental.pallas import tpu as pltpu
from jax.experimental.pallas import tpu_sc as plsc

_NW = 32
_CHUNK = 200
_PBLK = 5000
_SBLK = 1600


def _bf16_bits(w):
    r = w.astype(jnp.bfloat16).astype(jnp.float32)
    return jax.lax.shift_right_logical(
        jax.lax.bitcast_convert_type(r, jnp.int32), 16)


def _pack_body(w3_ref, w5_ref, w7_ref, out_ref):
    w3, w5, w7 = w3_ref[...], w5_ref[...], w7_ref[...]
    r, d = w3.shape
    lo = jnp.concatenate([w3, w5], axis=1)
    hi = jnp.concatenate([w7, jnp.zeros((r, d), jnp.float32)], axis=1)
    out_ref[...] = _bf16_bits(lo) | (_bf16_bits(hi) << 16)


def _pack_tables(w3, w5, w7):
    v, d = w3.shape
    grid = v // _PBLK
    return pl.pallas_call(
        _pack_body,
        grid=(grid,),
        in_specs=[pl.BlockSpec((_PBLK, d), lambda i: (i, 0))] * 3,
        out_specs=pl.BlockSpec((_PBLK, 2 * d), lambda i: (i, 0)),
        out_shape=jax.ShapeDtypeStruct((v, 2 * d), jnp.int32),
    )(w3, w5, w7)


def _sc_gather(table, idx):
    b = idx.shape[0]
    _, d = table.shape
    b_per_w = b // _NW
    n_chunks = b_per_w // _CHUNK
    n_pairs = n_chunks // 2
    mesh = plsc.VectorSubcoreMesh(core_axis_name="c", subcore_axis_name="s")

    @functools.partial(
        pl.kernel,
        out_type=jax.ShapeDtypeStruct((b, d), table.dtype),
        mesh=mesh,
        scratch_types=[
            pltpu.VMEM((_CHUNK,), jnp.int32),
            pltpu.VMEM((_CHUNK,), jnp.int32),
            pltpu.VMEM((_CHUNK, d), table.dtype),
            pltpu.VMEM((_CHUNK, d), table.dtype),
            pltpu.SemaphoreType.DMA,
            pltpu.SemaphoreType.DMA,
            pltpu.SemaphoreType.DMA,
            pltpu.SemaphoreType.DMA,
        ],
    )
    def gather_kernel(table_hbm, idx_hbm, out_hbm,
                      i0, i1, r0, r1, sg0, sg1, sw0, sw1):
        wid = jax.lax.axis_index("s") * 2 + jax.lax.axis_index("c")
        tile_base = wid * b_per_w

        @pl.loop(0, n_pairs)
        def _(p):
            base0 = tile_base + 2 * p * _CHUNK
            base1 = base0 + _CHUNK

            @pl.when(p > 0)
            def _():
                pltpu.make_async_copy(
                    r0, out_hbm.at[pl.ds(base0 - 2 * _CHUNK, _CHUNK)],
                    sw0).wait()
                pltpu.make_async_copy(
                    r1, out_hbm.at[pl.ds(base1 - 2 * _CHUNK, _CHUNK)],
                    sw1).wait()

            pltpu.sync_copy(idx_hbm.at[pl.ds(base0, _CHUNK)], i0)
            g0 = pltpu.async_copy(table_hbm.at[i0], r0, sg0)
            pltpu.sync_copy(idx_hbm.at[pl.ds(base1, _CHUNK)], i1)
            g1 = pltpu.async_copy(table_hbm.at[i1], r1, sg1)
            g0.wait()
            pltpu.async_copy(r0, out_hbm.at[pl.ds(base0, _CHUNK)], sw0)
            g1.wait()
            pltpu.async_copy(r1, out_hbm.at[pl.ds(base1, _CHUNK)], sw1)

        end0 = tile_base + (n_chunks - 2) * _CHUNK
        pltpu.make_async_copy(
            r0, out_hbm.at[pl.ds(end0, _CHUNK)], sw0).wait()
        pltpu.make_async_copy(
            r1, out_hbm.at[pl.ds(end0 + _CHUNK, _CHUNK)], sw1).wait()

    return gather_kernel(table, idx)


def _band_matrix(d, k):
    i = jnp.arange(d)
    band = (jnp.abs(i[:, None] - i[None, :]) <= (k - 1) // 2)
    return band.astype(jnp.float32) * (1.0 / k)


def _smooth_matrix(d):
    m = jnp.zeros((4 * d, 3 * d), jnp.float32)
    for b, k in enumerate((3, 5, 7)):
        m = m.at[b * d:(b + 1) * d, b * d:(b + 1) * d].set(_band_matrix(d, k))
    return m


def _strip_body(raw_ref, mlo_ref, mhi_ref, out_ref):
    raw = raw_ref[...]
    lo = jax.lax.bitcast_convert_type(raw << 16, jnp.float32)
    hi = jax.lax.bitcast_convert_type(raw & jnp.int32(-65536), jnp.float32)
    out_ref[...] = (
        jnp.dot(lo, mlo_ref[...], preferred_element_type=jnp.float32)
        + jnp.dot(hi, mhi_ref[...], preferred_element_type=jnp.float32))


def _smooth_compact(raw, mlo, mhi):
    b, dpak = raw.shape
    d_out = mlo.shape[1]
    nb = b // _SBLK
    m_spec = pl.BlockSpec((dpak, d_out), lambda j: (0, 0))
    return pl.pallas_call(
        _strip_body,
        grid=(nb,),
        in_specs=[pl.BlockSpec((_SBLK, dpak), lambda j: (j, 0)),
                  m_spec, m_spec],
        out_specs=pl.BlockSpec((_SBLK, d_out), lambda j: (j, 0)),
        out_shape=jax.ShapeDtypeStruct((b, d_out), jnp.float32),
    )(raw, mlo, mhi)


def kernel(x, W3, W5, W7):
    bsz, seq = x.shape
    v, d = W3.shape
    tpack = _pack_tables(W3, W5, W7)
    idx = x.reshape(-1).astype(jnp.int32)
    raw = _sc_gather(tpack, idx)
    m = _smooth_matrix(d)
    mlo, mhi = m[:2 * d], m[2 * d:]
    out = _smooth_compact(raw, mlo, mhi)
    return out.reshape(bsz, seq, 3 * d)


# speedup vs baseline: 1.9433x; 1.2386x over previous
import functools

import jax
import jax.numpy as jnp
from jax.experimental import pallas as pl
from jax.experimental.pallas import tpu as pltpu
from jax.experimental.pallas import tpu_sc as plsc

_NW = 32
_CHUNK = 200
_PBLK = 5000
_SBLK = 1600


def _bf16_bits(w):
    r = w.astype(jnp.bfloat16).astype(jnp.float32)
    return jax.lax.shift_right_logical(
        jax.lax.bitcast_convert_type(r, jnp.int32), 16)


def _pack_body(w3_ref, w5_ref, w7_ref, out_ref):
    w3, w5, w7 = w3_ref[...], w5_ref[...], w7_ref[...]
    r, d = w3.shape
    lo = jnp.concatenate([w3, w5], axis=1)
    hi = jnp.concatenate([w7, jnp.zeros((r, d), jnp.float32)], axis=1)
    out_ref[...] = _bf16_bits(lo) | (_bf16_bits(hi) << 16)


def _pack_tables(w3, w5, w7):
    v, d = w3.shape
    grid = v // _PBLK
    return pl.pallas_call(
        _pack_body,
        grid=(grid,),
        in_specs=[pl.BlockSpec((_PBLK, d), lambda i: (i, 0))] * 3,
        out_specs=pl.BlockSpec((_PBLK, 2 * d), lambda i: (i, 0)),
        out_shape=jax.ShapeDtypeStruct((v, 2 * d), jnp.int32),
    )(w3, w5, w7)


def _sc_gather(table, idx):
    b = idx.shape[0]
    _, d = table.shape
    b_per_w = b // _NW
    n_chunks = b_per_w // _CHUNK
    n_pairs = n_chunks // 2
    mesh = plsc.VectorSubcoreMesh(core_axis_name="c", subcore_axis_name="s")

    @functools.partial(
        pl.kernel,
        out_type=jax.ShapeDtypeStruct((b, d), table.dtype),
        mesh=mesh,
        scratch_types=[
            pltpu.VMEM((_CHUNK,), jnp.int32),
            pltpu.VMEM((_CHUNK,), jnp.int32),
            pltpu.VMEM((_CHUNK, d), table.dtype),
            pltpu.VMEM((_CHUNK, d), table.dtype),
            pltpu.SemaphoreType.DMA,
            pltpu.SemaphoreType.DMA,
            pltpu.SemaphoreType.DMA,
            pltpu.SemaphoreType.DMA,
        ],
    )
    def gather_kernel(table_hbm, idx_hbm, out_hbm,
                      i0, i1, r0, r1, sg0, sg1, sw0, sw1):
        wid = jax.lax.axis_index("s") * 2 + jax.lax.axis_index("c")
        tile_base = wid * b_per_w

        @pl.loop(0, n_pairs)
        def _(p):
            base0 = tile_base + 2 * p * _CHUNK
            base1 = base0 + _CHUNK

            @pl.when(p > 0)
            def _():
                pltpu.make_async_copy(
                    r0, out_hbm.at[pl.ds(base0 - 2 * _CHUNK, _CHUNK)],
                    sw0).wait()
                pltpu.make_async_copy(
                    r1, out_hbm.at[pl.ds(base1 - 2 * _CHUNK, _CHUNK)],
                    sw1).wait()

            pltpu.sync_copy(idx_hbm.at[pl.ds(base0, _CHUNK)], i0)
            g0 = pltpu.async_copy(table_hbm.at[i0], r0, sg0)
            pltpu.sync_copy(idx_hbm.at[pl.ds(base1, _CHUNK)], i1)
            g1 = pltpu.async_copy(table_hbm.at[i1], r1, sg1)
            g0.wait()
            pltpu.async_copy(r0, out_hbm.at[pl.ds(base0, _CHUNK)], sw0)
            g1.wait()
            pltpu.async_copy(r1, out_hbm.at[pl.ds(base1, _CHUNK)], sw1)

        end0 = tile_base + (n_chunks - 2) * _CHUNK
        pltpu.make_async_copy(
            r0, out_hbm.at[pl.ds(end0, _CHUNK)], sw0).wait()
        pltpu.make_async_copy(
            r1, out_hbm.at[pl.ds(end0 + _CHUNK, _CHUNK)], sw1).wait()

    return gather_kernel(table, idx)


def _band_matrix(d, k):
    i = jnp.arange(d)
    band = (jnp.abs(i[:, None] - i[None, :]) <= (k - 1) // 2)
    return band.astype(jnp.float32) * (1.0 / k)


def _smooth_matrix(d):
    m = jnp.zeros((4 * d, 3 * d), jnp.float32)
    for b, k in enumerate((3, 5, 7)):
        m = m.at[b * d:(b + 1) * d, b * d:(b + 1) * d].set(_band_matrix(d, k))
    return m


def _strip_body(raw_ref, mlo_ref, mhi_ref, out_ref):
    raw = raw_ref[...]
    lo = jax.lax.bitcast_convert_type(raw << 16, jnp.float32)
    hi = jax.lax.bitcast_convert_type(raw & jnp.int32(-65536), jnp.float32)
    res = (jnp.dot(lo, mlo_ref[...], preferred_element_type=jnp.float32)
           + jnp.dot(hi, mhi_ref[...], preferred_element_type=jnp.float32))
    bb, _, d_out = out_ref.shape
    out_ref[...] = res.reshape(bb, -1, d_out)


def _smooth_compact(raw, mlo, mhi, bsz, seq):
    _, dpak = raw.shape
    d_out = mlo.shape[1]
    bb = _SBLK // seq
    nb = bsz // bb
    m_spec = pl.BlockSpec((dpak, d_out), lambda j: (0, 0))
    return pl.pallas_call(
        _strip_body,
        grid=(nb,),
        in_specs=[pl.BlockSpec((bb * seq, dpak), lambda j: (j, 0)),
                  m_spec, m_spec],
        out_specs=pl.BlockSpec((bb, seq, d_out), lambda j: (j, 0, 0)),
        out_shape=jax.ShapeDtypeStruct((bsz, seq, d_out), jnp.float32),
    )(raw, mlo, mhi)


def kernel(x, W3, W5, W7):
    bsz, seq = x.shape
    v, d = W3.shape
    tpack = _pack_tables(W3, W5, W7)
    idx = x.reshape(-1).astype(jnp.int32)
    raw = _sc_gather(tpack, idx)
    m = _smooth_matrix(d)
    mlo, mhi = m[:2 * d], m[2 * d:]
    return _smooth_compact(raw, mlo, mhi, bsz, seq)
